# baseline (speedup 1.0000x reference)
import functools
import jax
import jax.numpy as jnp
from jax import lax
from jax.experimental import pallas as pl
from jax.experimental.pallas import tpu as pltpu
from jax.experimental.pallas import tpu_sc as plsc

NUM_NODES_K = 50000
EMBED_DIM_K = 128
BATCH_K = 8

TS_BATCHES = (0, 1, 2, 3, 4, 5)
CHUNK_N = 200
N_CHUNKS_TOT = NUM_NODES_K // CHUNK_N
N_TS_WORKERS = 30
TS_CHUNKS_PER_W = -(-N_CHUNKS_TOT // N_TS_WORKERS)
NBUF = 3

SP_BATCHES = (6, 7)
SP_CHUNK_N = 1000
SP_CHUNKS_PER_C = (NUM_NODES_K // 2) // SP_CHUNK_N
SP_NBUF = 2


def _sc_body(t_hbm, o_hbm,
             b0, b1, b2, i0, i1, i2, o0, o1, o2,
             sp0, sp1, spi0, spi1, spo0, spo1):
    cid_c = lax.axis_index("c")
    sid = lax.axis_index("s")

    bufs = [b0, b1, b2]
    isems = [i0, i1, i2]
    osems = [o0, o1, o2]
    wid = cid_c * (N_TS_WORKERS // 2) + (sid - 1)

    def rows_of(i):
        cid = wid + i * N_TS_WORKERS
        return cid, pl.ds(cid * CHUNK_N, CHUNK_N)

    def start_load(i):
        if i >= TS_CHUNKS_PER_W:
            return
        cid, rows = rows_of(i)

        @pl.when(cid < N_CHUNKS_TOT)
        def _():
            pltpu.async_copy(t_hbm.at[rows, :], bufs[i % NBUF],
                             isems[i % NBUF])

    def wait_load(i):
        cid, rows = rows_of(i)

        @pl.when(cid < N_CHUNKS_TOT)
        def _():
            pltpu.make_async_copy(t_hbm.at[rows, :], bufs[i % NBUF],
                                  isems[i % NBUF]).wait()

    def start_stores(i):
        cid, rows = rows_of(i)

        @pl.when(cid < N_CHUNKS_TOT)
        def _():
            for b in TS_BATCHES:
                pltpu.async_copy(bufs[i % NBUF], o_hbm.at[b, rows, :],
                                 osems[i % NBUF])

    def wait_stores(i):
        if i < 0 or i >= TS_CHUNKS_PER_W:
            return
        cid, rows = rows_of(i)

        @pl.when(cid < N_CHUNKS_TOT)
        def _():
            for b in TS_BATCHES:
                pltpu.make_async_copy(bufs[i % NBUF], o_hbm.at[b, rows, :],
                                      osems[i % NBUF]).wait()

    @pl.when(sid > 0)
    def _():
        for i in range(NBUF - 1):
            start_load(i)
        for i in range(TS_CHUNKS_PER_W):
            wait_load(i)
            start_stores(i)
            wait_stores(i - 1)
            start_load(i + NBUF - 1)
        wait_stores(TS_CHUNKS_PER_W - 1)

    spbufs = [sp0, sp1]
    spisems = [spi0, spi1]
    sposems = [spo0, spo1]
    base_row = cid_c * (NUM_NODES_K // 2)

    def sp_rows_of(i):
        return pl.ds(base_row + i * SP_CHUNK_N, SP_CHUNK_N)

    def sp_start_load(i):
        if i >= SP_CHUNKS_PER_C:
            return
        pltpu.async_copy(t_hbm.at[sp_rows_of(i), :], spbufs[i % SP_NBUF],
                         spisems[i % SP_NBUF])

    def sp_wait_load(i):
        pltpu.make_async_copy(t_hbm.at[sp_rows_of(i), :], spbufs[i % SP_NBUF],
                              spisems[i % SP_NBUF]).wait()

    def sp_start_stores(i):
        for b in SP_BATCHES:
            pltpu.async_copy(spbufs[i % SP_NBUF], o_hbm.at[b, sp_rows_of(i), :],
                             sposems[i % SP_NBUF])

    def sp_wait_stores(i):
        if i < 0 or i >= SP_CHUNKS_PER_C:
            return
        for b in SP_BATCHES:
            pltpu.make_async_copy(spbufs[i % SP_NBUF],
                                  o_hbm.at[b, sp_rows_of(i), :],
                                  sposems[i % SP_NBUF]).wait()

    @pl.when(sid == 0)
    def _():
        sp_start_load(0)
        for i in range(SP_CHUNKS_PER_C):
            sp_wait_load(i)
            sp_start_stores(i)
            sp_wait_stores(i - 1)
            sp_start_load(i + SP_NBUF - 1)
        sp_wait_stores(SP_CHUNKS_PER_C - 1)


def kernel(table, batch_size):
    del batch_size
    mesh = plsc.VectorSubcoreMesh(core_axis_name="c", subcore_axis_name="s")
    run = functools.partial(
        pl.kernel,
        mesh=mesh,
        out_type=jax.ShapeDtypeStruct((BATCH_K, NUM_NODES_K, EMBED_DIM_K),
                                      jnp.float32),
        scratch_types=(
            [pltpu.VMEM((CHUNK_N, EMBED_DIM_K), jnp.float32)] * NBUF
            + [pltpu.SemaphoreType.DMA] * (2 * NBUF)
            + [pltpu.VMEM_SHARED((SP_CHUNK_N, EMBED_DIM_K), jnp.float32)] * SP_NBUF
            + [pltpu.SemaphoreType.DMA] * (2 * SP_NBUF)
        ),
    )(_sc_body)
    return run(table)


# speedup vs baseline: 1.1770x; 1.1770x over previous
import functools
import jax
import jax.numpy as jnp
from jax import lax
from jax.experimental import pallas as pl
from jax.experimental.pallas import tpu as pltpu
from jax.experimental.pallas import tpu_sc as plsc

NUM_NODES_K = 50000
EMBED_DIM_K = 128
BATCH_K = 8
CHUNK_N = 400
N_CHUNKS_TOT = NUM_NODES_K // CHUNK_N
N_WORKERS = 32
CHUNKS_PER_W = -(-N_CHUNKS_TOT // N_WORKERS)
NBUF = 2


def _sc_body(t_hbm, o_hbm, b0, b1, i0, i1, o0, o1):
    bufs = [b0, b1]
    isems = [i0, i1]
    osems = [o0, o1]
    wid = lax.axis_index("c") * 16 + lax.axis_index("s")

    def rows_of(i):
        cid = wid + i * N_WORKERS
        return cid, pl.ds(cid * CHUNK_N, CHUNK_N)

    def start_load(i):
        if i >= CHUNKS_PER_W:
            return
        cid, rows = rows_of(i)

        @pl.when(cid < N_CHUNKS_TOT)
        def _():
            pltpu.async_copy(t_hbm.at[rows, :], bufs[i % NBUF],
                             isems[i % NBUF])

    def wait_load(i):
        cid, rows = rows_of(i)

        @pl.when(cid < N_CHUNKS_TOT)
        def _():
            pltpu.make_async_copy(t_hbm.at[rows, :], bufs[i % NBUF],
                                  isems[i % NBUF]).wait()

    def start_stores(i):
        cid, rows = rows_of(i)

        @pl.when(cid < N_CHUNKS_TOT)
        def _():
            for b in range(BATCH_K):
                pltpu.async_copy(bufs[i % NBUF], o_hbm.at[b, rows, :],
                                 osems[i % NBUF])

    def wait_stores(i):
        if i < 0 or i >= CHUNKS_PER_W:
            return
        cid, rows = rows_of(i)

        @pl.when(cid < N_CHUNKS_TOT)
        def _():
            for b in range(BATCH_K):
                pltpu.make_async_copy(bufs[i % NBUF], o_hbm.at[b, rows, :],
                                      osems[i % NBUF]).wait()

    for i in range(NBUF - 1):
        start_load(i)
    for i in range(CHUNKS_PER_W):
        wait_load(i)
        start_stores(i)
        wait_stores(i - 1)
        start_load(i + NBUF - 1)
    wait_stores(CHUNKS_PER_W - 1)


def kernel(table, batch_size):
    del batch_size
    mesh = plsc.VectorSubcoreMesh(core_axis_name="c", subcore_axis_name="s")
    run = functools.partial(
        pl.kernel,
        mesh=mesh,
        out_type=jax.ShapeDtypeStruct((BATCH_K, NUM_NODES_K, EMBED_DIM_K),
                                      jnp.float32),
        scratch_types=(
            [pltpu.VMEM((CHUNK_N, EMBED_DIM_K), jnp.float32)] * NBUF
            + [pltpu.SemaphoreType.DMA] * (2 * NBUF)
        ),
    )(_sc_body)
    return run(table)
